# baseline (speedup 1.0000x reference)
import functools

import jax
import jax.numpy as jnp
from jax import lax
from jax.experimental import pallas as pl
from jax.experimental.pallas import tpu as pltpu

N = 20000
THRESH = 0.5
MAX_PROPOSALS = 1000
B = 256
NP = 20224
NBLK = NP // B
OUTC = 1024

_ROWDOT = functools.partial(
    lax.dot_general,
    dimension_numbers=(((1,), (0,)), ((), ())),
    precision=lax.Precision.DEFAULT,
    preferred_element_type=jnp.float32,
)
_TRDOT = functools.partial(
    lax.dot_general,
    dimension_numbers=(((1,), (1,)), ((), ())),
    precision=lax.Precision.HIGHEST,
    preferred_element_type=jnp.float32,
)
_DOT_HI = functools.partial(
    lax.dot_general,
    dimension_numbers=(((1,), (0,)), ((), ())),
    precision=lax.Precision.HIGHEST,
    preferred_element_type=jnp.float32,
)


def _make_nms_kernel(nblk):
  def _nms_kernel(bT_ref, outT_ref, alive_ref, colsC_ref):
    f32 = jnp.float32
    outT_ref[...] = jnp.zeros((8, OUTC), f32)

    row_i = lax.broadcasted_iota(jnp.int32, (B, B), 0)
    col_i = lax.broadcasted_iota(jnp.int32, (B, B), 1)
    eye = (row_i == col_i).astype(f32)
    usup_strict = (row_i < col_i).astype(f32)
    utri_incl = (row_i <= col_i).astype(f32)
    out_iota = lax.broadcasted_iota(jnp.int32, (B, OUTC), 1)

    def tr(v):
        return _TRDOT(eye, v)

    def iou_tile(ci, rj):
        x1r = colsC_ref[pl.ds(rj * B, B), 0:1]
        y1r = colsC_ref[pl.ds(rj * B, B), 1:2]
        x2r = colsC_ref[pl.ds(rj * B, B), 2:3]
        y2r = colsC_ref[pl.ds(rj * B, B), 3:4]
        x1c = bT_ref[0:1, pl.ds(ci * B, B)]
        y1c = bT_ref[1:2, pl.ds(ci * B, B)]
        x2c = bT_ref[2:3, pl.ds(ci * B, B)]
        y2c = bT_ref[3:4, pl.ds(ci * B, B)]
        arear = (x2r - x1r) * (y2r - y1r)
        areac = (x2c - x1c) * (y2c - y1c)
        w = jnp.maximum(jnp.minimum(x2c, x2r) - jnp.maximum(x1c, x1r), 0.0)
        h = jnp.maximum(jnp.minimum(y2c, y2r) - jnp.maximum(y1c, y1r), 0.0)
        inter = w * h
        return inter / (areac + arear - inter + 1e-9)

    def cond(state):
        i, count = state
        return jnp.logical_and(i < nblk, count < MAX_PROPOSALS)

    def body(state):
        i, count = state

        colsC_ref[pl.ds(i * B, B), 0:1] = tr(bT_ref[0:1, pl.ds(i * B, B)])
        colsC_ref[pl.ds(i * B, B), 1:2] = tr(bT_ref[1:2, pl.ds(i * B, B)])
        colsC_ref[pl.ds(i * B, B), 2:3] = tr(bT_ref[2:3, pl.ds(i * B, B)])
        colsC_ref[pl.ds(i * B, B), 3:4] = tr(bT_ref[3:4, pl.ds(i * B, B)])

        def jbody(j, supp):
            m = (iou_tile(i, j) > THRESH).astype(f32)
            aj = alive_ref[0:1, pl.ds(j * B, B)]
            return jnp.maximum(supp, _ROWDOT(aj, m))

        supp = lax.fori_loop(0, i, jbody, jnp.zeros((1, B), f32))
        real = bT_ref[5:6, pl.ds(i * B, B)]
        elig = jnp.logical_and(real > 0.5, supp < 0.5)

        m_self = (iou_tile(i, i) > THRESH).astype(f32) * usup_strict

        def fcond(c):
            return c[1]

        def fbody(c):
            a, _ = c
            s = _ROWDOT(a, m_self)
            anew = jnp.where(jnp.logical_and(elig, s < 0.5), 1.0, 0.0)
            return anew, jnp.any(anew != a)

        a0 = elig.astype(f32)
        aliv, _ = lax.while_loop(fcond, fbody, (a0, jnp.bool_(True)))
        alive_ref[0:1, pl.ds(i * B, B)] = aliv

        ranks = _ROWDOT(aliv, utri_incl) - 1.0 + count.astype(f32)
        rankmask = jnp.where(aliv > 0.5, ranks, -1.0)
        rk = tr(rankmask).astype(jnp.int32)
        sel = (out_iota == rk).astype(f32)
        dataT = bT_ref[:, pl.ds(i * B, B)]
        outT_ref[...] += _DOT_HI(dataT, sel)
        return i + 1, count + jnp.sum(aliv).astype(jnp.int32)

    _, fcount = lax.while_loop(cond, body, (jnp.int32(0), jnp.int32(0)))
    outT_ref[7:8, :] = jnp.full((1, OUTC), fcount.astype(f32))

  return _nms_kernel


def _run_nms(rows, np_):
    outT = pl.pallas_call(
        _make_nms_kernel(np_ // B),
        out_shape=jax.ShapeDtypeStruct((8, OUTC), jnp.float32),
        scratch_shapes=[pltpu.VMEM((1, np_), jnp.float32),
                        pltpu.VMEM((np_, 4), jnp.float32)],
    )(rows)
    return outT[:5, :MAX_PROPOSALS].T, outT[7, 0]


K_FAST = 2048


def kernel(boxes, scores):
    v, idx = lax.top_k(scores, K_FAST)
    bsel = boxes[idx]
    ones_k = jnp.ones((1, K_FAST), jnp.float32)
    rows_fast = jnp.concatenate(
        [bsel[:, 0][None, :], bsel[:, 1][None, :],
         bsel[:, 2][None, :], bsel[:, 3][None, :],
         v[None, :], ones_k, 0.0 * ones_k, 0.0 * ones_k], axis=0)
    out_fast, cnt = _run_nms(rows_fast, K_FAST)

    def fast_fn(_):
        return out_fast

    def full_fn(_):
        neg = -scores
        _, sx1, sy1, sx2, sy2, scores_s = lax.sort(
            (neg, boxes[:, 0], boxes[:, 1], boxes[:, 2], boxes[:, 3],
             scores), num_keys=1, is_stable=True)
        pad = NP - N

        def row(vv, fill):
            return jnp.concatenate(
                [vv, jnp.full((pad,), fill, jnp.float32)])[None, :]

        zero = jnp.zeros((1, NP), jnp.float32)
        rows = jnp.concatenate(
            [row(sx1, -1e6), row(sy1, -1e6), row(sx2, -1e6),
             row(sy2, -1e6), row(scores_s, -3e38),
             row(jnp.ones((N,), jnp.float32), 0.0), zero, zero], axis=0)
        out_full, _ = _run_nms(rows, NP)
        return out_full

    return lax.cond(cnt >= MAX_PROPOSALS, fast_fn, full_fn, None)
